# baseline (speedup 1.0000x reference)
import jax
import jax.numpy as jnp
from jax import lax
from jax.experimental import pallas as pl
from jax.experimental.pallas import tpu as pltpu
from jax.experimental.pallas import tpu_sc as plsc

VOCAB = 1000000
DIM = 32
BAGS = 106496
N = 425984

NC = 2
NS = 16
NW = NC * NS
NB = BAGS // NW
PASSES = 1
NBP = NB // PASSES
C = 128
K = 4
BLK = K * C
OFF_WIN = 3344
SC_BAGS = NS * NBP
DUMMY = SC_BAGS
ACC_ROWS = SC_BAGS + C

OFF_PAD = BAGS + OFF_WIN
IDX_PAD = N + 4 * BLK

_BITS = (2048, 1024, 512, 256, 128, 64, 32, 16, 8, 4, 2, 1)


def _sc_body(idx_hbm, off_hbm, table_hbm, out_hbm,
             off_v, idx_v, seg_v, rows_v, acc0,
             sem_i, sem_g, sem_s, sem_z, sem_e):
  c = lax.axis_index("c")
  s = lax.axis_index("s")
  wid = c * NS + s
  lane = lax.iota(jnp.int32, 16)
  accs = [acc0]

  def _zrow(i, _):
    rows_v[0, i, pl.ds(0, 16)] = jnp.zeros((16,), jnp.float32)
    rows_v[0, i, pl.ds(16, 16)] = jnp.zeros((16,), jnp.float32)
    return 0

  for pp in range(PASSES):
    acc = accs[pp]
    b0 = wid * NB + pp * NBP
    sc_b0 = s * NBP

    lax.fori_loop(0, 128, _zrow, 0)
    for k in range(NBP // 128):
      pltpu.async_copy(rows_v.at[0], acc.at[pl.ds(sc_b0 + k * 128, 128)],
                       sem_z)

    @pl.when(s == 0)
    def _():
      pltpu.async_copy(rows_v.at[0], acc.at[pl.ds(DUMMY, 128)], sem_z)

    pltpu.sync_copy(off_hbm.at[pl.ds(b0, OFF_WIN)], off_v)
    start = jnp.sum(jnp.where(lane == 0, off_v[pl.ds(0, 16)], 0))
    end = jnp.sum(jnp.where(lane == 0, off_v[pl.ds(NBP, 16)], 0))
    base = (start // 8) * 8
    nblk = (end - base + (BLK - 1)) // BLK

    @pl.when(nblk > 0)
    def _():
      pltpu.async_copy(idx_hbm.at[pl.ds(base, BLK)], idx_v.at[0], sem_i)

    for k in range(NBP // 128):
      pltpu.make_async_copy(
          rows_v.at[0], acc.at[pl.ds(sc_b0 + k * 128, 128)], sem_z).wait()

    @pl.when(s == 0)
    def _():
      pltpu.make_async_copy(
          rows_v.at[0], acc.at[pl.ds(DUMMY, 128)], sem_z).wait()

    def blk_body(i, carry):
      d = i % 2
      p0 = base + i * BLK
      pltpu.make_async_copy(
          idx_hbm.at[pl.ds(p0, BLK)], idx_v.at[d], sem_i).wait()
      gd = [
          pltpu.async_copy(
              table_hbm.at[idx_v.at[d, pl.ds(j * C, C)]], rows_v.at[j],
              sem_g)
          for j in range(K)
      ]
      pltpu.async_copy(
          idx_hbm.at[pl.ds(p0 + BLK, BLK)], idx_v.at[1 - d], sem_i)

      def seg_body(j, carry2):
        q0 = p0 + j * C
        for g in range(C // 16):
          p = q0 + g * 16 + lane
          cnt = jnp.zeros((16,), jnp.int32)
          for bit in _BITS:
            nxt = cnt + bit
            probe = jnp.minimum(nxt - 1, OFF_WIN - 1)
            vals = plsc.load_gather(off_v, [probe])
            cnt = jnp.where(vals <= p, nxt, cnt)
          segl = cnt - 1
          valid = (p >= start) & (p < end)
          seg = jnp.where(valid, sc_b0 + segl, DUMMY)
          seg_v[j, pl.ds(g * 16, 16)] = seg
        return carry2

      lax.fori_loop(0, K, seg_body, 0)

      for j in range(K):
        gd[j].wait()
      sd = [
          pltpu.async_copy(rows_v.at[j], acc.at[seg_v.at[j]], sem_s,
                           add=True)
          for j in range(K)
      ]
      for j in range(K):
        sd[j].wait()
      return carry

    lax.fori_loop(0, nblk, blk_body, 0)

    @pl.when(nblk > 0)
    def _():
      pltpu.make_async_copy(
          idx_hbm.at[pl.ds(base, BLK)], idx_v.at[0], sem_i).wait()

    pltpu.async_copy(acc.at[pl.ds(sc_b0, NBP)], out_hbm.at[pl.ds(b0, NBP)],
                     sem_e)

  for pp in range(PASSES):
    b0 = wid * NB + pp * NBP
    sc_b0 = s * NBP
    pltpu.make_async_copy(
        accs[pp].at[pl.ds(sc_b0, NBP)], out_hbm.at[pl.ds(b0, NBP)],
        sem_e).wait()


@jax.jit
def _run(idx_pad, off_pad, table):
  mesh = plsc.VectorSubcoreMesh(core_axis_name="c", subcore_axis_name="s")
  f = pl.kernel(
      _sc_body,
      out_type=jax.ShapeDtypeStruct((BAGS, DIM), jnp.float32),
      mesh=mesh,
      scratch_types=[
          pltpu.VMEM((OFF_WIN,), jnp.int32),
          pltpu.VMEM((2, BLK), jnp.int32),
          pltpu.VMEM((K, C), jnp.int32),
          pltpu.VMEM((K, C, DIM), jnp.float32),
          pltpu.VMEM_SHARED((ACC_ROWS, DIM), jnp.float32),
          pltpu.SemaphoreType.DMA,
          pltpu.SemaphoreType.DMA,
          pltpu.SemaphoreType.DMA,
          pltpu.SemaphoreType.DMA,
          pltpu.SemaphoreType.DMA,
      ],
      compiler_params=pltpu.CompilerParams(
          needs_layout_passes=False, use_tc_tiling_on_sc=False),
  )
  return f(idx_pad, off_pad, table)


def kernel(indices, offsets, table):
  idx32 = indices.astype(jnp.int32)
  off32 = offsets.astype(jnp.int32)
  idx_pad = jnp.concatenate(
      [idx32, jnp.zeros((IDX_PAD - N,), jnp.int32)])
  off_pad = jnp.concatenate(
      [off32, jnp.full((OFF_PAD - (BAGS + 1),), N, jnp.int32)])
  return _run(idx_pad, off_pad, table)


# speedup vs baseline: 1.0032x; 1.0032x over previous
import jax
import jax.numpy as jnp
from jax import lax
from jax.experimental import pallas as pl
from jax.experimental.pallas import tpu as pltpu
from jax.experimental.pallas import tpu_sc as plsc

VOCAB = 1000000
DIM = 32
BAGS = 106496
N = 425984

NC = 2
NS = 16
NW = NC * NS
NB = BAGS // NW
PASSES = 2
NBP = NB // PASSES
C = 128
K = 4
BLK = K * C
OFF_WIN = 1680
SC_BAGS = NS * NBP
DUMMY = SC_BAGS
ACC_ROWS = SC_BAGS + C

OFF_PAD = BAGS + OFF_WIN
IDX_PAD = N + 4 * BLK

_BITS = (1024, 512, 256, 128, 64, 32, 16, 8, 4, 2, 1)


def _sc_body(idx_hbm, off_hbm, table_hbm, out_hbm,
             off_v, idx_v, seg_v, rows_v, acc0, acc1,
             sem_i, sem_g, sem_s, sem_z, sem_e):
  c = lax.axis_index("c")
  s = lax.axis_index("s")
  wid = c * NS + s
  lane = lax.iota(jnp.int32, 16)
  accs = [acc0, acc1]

  def _zrow(i, _):
    rows_v[0, i, pl.ds(0, 16)] = jnp.zeros((16,), jnp.float32)
    rows_v[0, i, pl.ds(16, 16)] = jnp.zeros((16,), jnp.float32)
    return 0

  for pp in range(PASSES):
    acc = accs[pp]
    b0 = wid * NB + pp * NBP
    sc_b0 = s * NBP

    lax.fori_loop(0, 128, _zrow, 0)
    for k in range(NBP // 128):
      pltpu.async_copy(rows_v.at[0], acc.at[pl.ds(sc_b0 + k * 128, 128)],
                       sem_z)

    @pl.when(s == 0)
    def _():
      pltpu.async_copy(rows_v.at[0], acc.at[pl.ds(DUMMY, 128)], sem_z)

    pltpu.sync_copy(off_hbm.at[pl.ds(b0, OFF_WIN)], off_v)
    start = jnp.sum(jnp.where(lane == 0, off_v[pl.ds(0, 16)], 0))
    end = jnp.sum(jnp.where(lane == 0, off_v[pl.ds(NBP, 16)], 0))
    base = (start // 8) * 8
    nblk = (end - base + (BLK - 1)) // BLK

    @pl.when(nblk > 0)
    def _():
      pltpu.async_copy(idx_hbm.at[pl.ds(base, BLK)], idx_v.at[0], sem_i)

    for k in range(NBP // 128):
      pltpu.make_async_copy(
          rows_v.at[0], acc.at[pl.ds(sc_b0 + k * 128, 128)], sem_z).wait()

    @pl.when(s == 0)
    def _():
      pltpu.make_async_copy(
          rows_v.at[0], acc.at[pl.ds(DUMMY, 128)], sem_z).wait()

    def blk_body(i, carry):
      d = i % 2
      p0 = base + i * BLK
      pltpu.make_async_copy(
          idx_hbm.at[pl.ds(p0, BLK)], idx_v.at[d], sem_i).wait()
      gd = [
          pltpu.async_copy(
              table_hbm.at[idx_v.at[d, pl.ds(j * C, C)]], rows_v.at[j],
              sem_g)
          for j in range(K)
      ]
      pltpu.async_copy(
          idx_hbm.at[pl.ds(p0 + BLK, BLK)], idx_v.at[1 - d], sem_i)

      def seg_body(j, carry2):
        q0 = p0 + j * C
        for g in range(C // 16):
          p = q0 + g * 16 + lane
          cnt = jnp.zeros((16,), jnp.int32)
          for bit in _BITS:
            nxt = cnt + bit
            probe = jnp.minimum(nxt - 1, OFF_WIN - 1)
            vals = plsc.load_gather(off_v, [probe])
            cnt = jnp.where(vals <= p, nxt, cnt)
          segl = cnt - 1
          valid = (p >= start) & (p < end)
          seg = jnp.where(valid, sc_b0 + segl, DUMMY)
          seg_v[j, pl.ds(g * 16, 16)] = seg
        return carry2

      lax.fori_loop(0, K, seg_body, 0)

      for j in range(K):
        gd[j].wait()
      sd = [
          pltpu.async_copy(rows_v.at[j], acc.at[seg_v.at[j]], sem_s,
                           add=True)
          for j in range(K)
      ]
      for j in range(K):
        sd[j].wait()
      return carry

    lax.fori_loop(0, nblk, blk_body, 0)

    @pl.when(nblk > 0)
    def _():
      pltpu.make_async_copy(
          idx_hbm.at[pl.ds(base, BLK)], idx_v.at[0], sem_i).wait()

    pltpu.async_copy(acc.at[pl.ds(sc_b0, NBP)], out_hbm.at[pl.ds(b0, NBP)],
                     sem_e)

  for pp in range(PASSES):
    b0 = wid * NB + pp * NBP
    sc_b0 = s * NBP
    pltpu.make_async_copy(
        accs[pp].at[pl.ds(sc_b0, NBP)], out_hbm.at[pl.ds(b0, NBP)],
        sem_e).wait()


@jax.jit
def _run(idx_pad, off_pad, table):
  mesh = plsc.VectorSubcoreMesh(core_axis_name="c", subcore_axis_name="s")
  f = pl.kernel(
      _sc_body,
      out_type=jax.ShapeDtypeStruct((BAGS, DIM), jnp.float32),
      mesh=mesh,
      scratch_types=[
          pltpu.VMEM((OFF_WIN,), jnp.int32),
          pltpu.VMEM((2, BLK), jnp.int32),
          pltpu.VMEM((K, C), jnp.int32),
          pltpu.VMEM((K, C, DIM), jnp.float32),
          pltpu.VMEM_SHARED((ACC_ROWS, DIM), jnp.float32),
          pltpu.VMEM_SHARED((ACC_ROWS, DIM), jnp.float32),
          pltpu.SemaphoreType.DMA,
          pltpu.SemaphoreType.DMA,
          pltpu.SemaphoreType.DMA,
          pltpu.SemaphoreType.DMA,
          pltpu.SemaphoreType.DMA,
      ],
      compiler_params=pltpu.CompilerParams(
          needs_layout_passes=False, use_tc_tiling_on_sc=False),
  )
  return f(idx_pad, off_pad, table)


def kernel(indices, offsets, table):
  idx32 = indices.astype(jnp.int32)
  off32 = offsets.astype(jnp.int32)
  idx_pad = jnp.concatenate(
      [idx32, jnp.zeros((IDX_PAD - N,), jnp.int32)])
  off_pad = jnp.concatenate(
      [off32, jnp.full((OFF_PAD - (BAGS + 1),), N, jnp.int32)])
  return _run(idx_pad, off_pad, table)
